# baseline (device time: 103519 ns/iter reference)
import jax
import jax.numpy as jnp
from jax import lax
from jax.experimental import pallas as pl
from jax.experimental.pallas import tpu as pltpu

N_DEV = 4


def _gelu(z):
    return 0.5 * z * (1.0 + jnp.tanh(0.7978845608 * (z + 0.044715 * z * z * z)))


def kernel(A, B):
    M, _ = A.shape
    _, N = B.shape
    Q = M // N_DEV

    def body(a_ref, b_ref, out_ref, part_ref,
             rs_send_buf, rs_recv_buf, ag_my_buf, ag_hop_buf,
             rs_send_sems, rs_recv_sems, ag_send_sems, ag_recv_sems):
        i = lax.axis_index("i")
        right = lax.rem(i + 1, N_DEV)
        left = lax.rem(i + (N_DEV - 1), N_DEV)

        barrier = pltpu.get_barrier_semaphore()
        for nbr in (left, right):
            pl.semaphore_signal(
                barrier, inc=1,
                device_id=(nbr,), device_id_type=pl.DeviceIdType.MESH,
            )
        pl.semaphore_wait(barrier, 2)

        part_ref[...] = jnp.dot(
            a_ref[...].astype(jnp.bfloat16),
            b_ref[...].astype(jnp.bfloat16),
            preferred_element_type=jnp.float32,
        )

        rs_send_buf[0] = part_ref[pl.ds(i * Q, Q), :].astype(jnp.bfloat16)
        reduced = None
        for s in range(N_DEV - 1):
            rdma = pltpu.make_async_remote_copy(
                src_ref=rs_send_buf.at[s],
                dst_ref=rs_recv_buf.at[s],
                send_sem=rs_send_sems.at[s],
                recv_sem=rs_recv_sems.at[s],
                device_id=(right,),
                device_id_type=pl.DeviceIdType.MESH,
            )
            rdma.start()
            rdma.wait()
            r = lax.rem(i + (2 * N_DEV - s - 1), N_DEV)
            summed = (rs_recv_buf[s].astype(jnp.float32)
                      + part_ref[pl.ds(r * Q, Q), :])
            if s < N_DEV - 2:
                rs_send_buf[s + 1] = summed.astype(jnp.bfloat16)
            else:
                reduced = summed

        q = lax.rem(i + 1, N_DEV)
        g = _gelu(reduced)
        out_ref[pl.ds(q * Q, Q), :] = g
        ag_my_buf[...] = g.astype(jnp.bfloat16)

        for h in range(N_DEV - 1):
            src = ag_my_buf if h == 0 else ag_hop_buf.at[h - 1]
            rdma = pltpu.make_async_remote_copy(
                src_ref=src,
                dst_ref=ag_hop_buf.at[h],
                send_sem=ag_send_sems.at[h],
                recv_sem=ag_recv_sems.at[h],
                device_id=(right,),
                device_id_type=pl.DeviceIdType.MESH,
            )
            rdma.start()
            rdma.wait()
            o = lax.rem(i + (N_DEV - h), N_DEV)
            out_ref[pl.ds(o * Q, Q), :] = ag_hop_buf[h].astype(jnp.float32)

    return pl.pallas_call(
        body,
        out_shape=jax.ShapeDtypeStruct((M, N), jnp.float32),
        in_specs=[
            pl.BlockSpec(memory_space=pltpu.VMEM),
            pl.BlockSpec(memory_space=pltpu.VMEM),
        ],
        out_specs=pl.BlockSpec(memory_space=pltpu.VMEM),
        scratch_shapes=[
            pltpu.VMEM((M, N), jnp.float32),
            pltpu.VMEM((N_DEV - 1, Q, N), jnp.bfloat16),
            pltpu.VMEM((N_DEV - 1, Q, N), jnp.bfloat16),
            pltpu.VMEM((Q, N), jnp.bfloat16),
            pltpu.VMEM((N_DEV - 1, Q, N), jnp.bfloat16),
            pltpu.SemaphoreType.DMA((N_DEV - 1,)),
            pltpu.SemaphoreType.DMA((N_DEV - 1,)),
            pltpu.SemaphoreType.DMA((N_DEV - 1,)),
            pltpu.SemaphoreType.DMA((N_DEV - 1,)),
        ],
        compiler_params=pltpu.CompilerParams(collective_id=0),
    )(A, B)


# device time: 64988 ns/iter; 1.5929x vs baseline; 1.5929x over previous
import jax
import jax.numpy as jnp
from jax import lax
from jax.experimental import pallas as pl
from jax.experimental.pallas import tpu as pltpu

N_DEV = 4


def _gelu(z):
    return 0.5 * z * (1.0 + jnp.tanh(0.7978845608 * (z + 0.044715 * z * z * z)))


def kernel(A, B):
    M, _ = A.shape
    _, N = B.shape
    Q = M // N_DEV
    H = Q // 2

    def body(a_ref, b_ref, out_ref, part_ref,
             cw_send, cw_recv, ccw_send, ccw_recv,
             cw_ag_my, cw_ag, ccw_ag_my, ccw_ag,
             cw_rs_ssem, cw_rs_rsem, ccw_rs_ssem, ccw_rs_rsem,
             cw_ag_ssem, cw_ag_rsem, ccw_ag_ssem, ccw_ag_rsem):
        i = lax.axis_index("i")
        right = lax.rem(i + 1, N_DEV)
        left = lax.rem(i + (N_DEV - 1), N_DEV)

        def chunk_top(c):
            return part_ref[pl.ds(c * Q, H), :]

        def chunk_bot(c):
            return part_ref[pl.ds(c * Q + H, H), :]

        barrier = pltpu.get_barrier_semaphore()
        for nbr in (left, right):
            pl.semaphore_signal(
                barrier, inc=1,
                device_id=(nbr,), device_id_type=pl.DeviceIdType.MESH,
            )
        pl.semaphore_wait(barrier, 2)

        part_ref[...] = jnp.dot(
            a_ref[...].astype(jnp.bfloat16),
            b_ref[...].astype(jnp.bfloat16),
            preferred_element_type=jnp.float32,
        )

        send_waits = []

        def rs_rdma(s, direction):
            if direction == "cw":
                return pltpu.make_async_remote_copy(
                    src_ref=cw_send.at[s], dst_ref=cw_recv.at[s],
                    send_sem=cw_rs_ssem.at[s], recv_sem=cw_rs_rsem.at[s],
                    device_id=(right,), device_id_type=pl.DeviceIdType.MESH,
                )
            return pltpu.make_async_remote_copy(
                src_ref=ccw_send.at[s], dst_ref=ccw_recv.at[s],
                send_sem=ccw_rs_ssem.at[s], recv_sem=ccw_rs_rsem.at[s],
                device_id=(left,), device_id_type=pl.DeviceIdType.MESH,
            )

        cw_send[0] = chunk_top(i).astype(jnp.bfloat16)
        ccw_send[0] = chunk_bot(i).astype(jnp.bfloat16)
        cw0 = rs_rdma(0, "cw")
        ccw0 = rs_rdma(0, "ccw")
        cw0.start()
        ccw0.start()
        send_waits += [cw0, ccw0]

        cw_rdmas = [cw0]
        ccw_rdmas = [ccw0]
        reduced_top = None
        reduced_bot = None
        for s in range(N_DEV - 1):
            cw_rdmas[s].wait_recv()
            r = lax.rem(i + (2 * N_DEV - s - 1), N_DEV)
            summed = cw_recv[s].astype(jnp.float32) + chunk_top(r)
            if s < N_DEV - 2:
                cw_send[s + 1] = summed.astype(jnp.bfloat16)
                nxt = rs_rdma(s + 1, "cw")
                nxt.start()
                cw_rdmas.append(nxt)
                send_waits.append(nxt)
            else:
                reduced_top = summed

            ccw_rdmas[s].wait_recv()
            r2 = lax.rem(i + s + 1, N_DEV)
            summed2 = ccw_recv[s].astype(jnp.float32) + chunk_bot(r2)
            if s < N_DEV - 2:
                ccw_send[s + 1] = summed2.astype(jnp.bfloat16)
                nxt = rs_rdma(s + 1, "ccw")
                nxt.start()
                ccw_rdmas.append(nxt)
                send_waits.append(nxt)
            else:
                reduced_bot = summed2

        q_cw = lax.rem(i + 1, N_DEV)
        q_ccw = lax.rem(i + (N_DEV - 1), N_DEV)
        g_top = _gelu(reduced_top)
        cw_ag_my[...] = g_top.astype(jnp.bfloat16)

        def ag_rdma(h, direction):
            if direction == "cw":
                src = cw_ag_my if h == 0 else cw_ag.at[h - 1]
                return pltpu.make_async_remote_copy(
                    src_ref=src, dst_ref=cw_ag.at[h],
                    send_sem=cw_ag_ssem.at[h], recv_sem=cw_ag_rsem.at[h],
                    device_id=(right,), device_id_type=pl.DeviceIdType.MESH,
                )
            src = ccw_ag_my if h == 0 else ccw_ag.at[h - 1]
            return pltpu.make_async_remote_copy(
                src_ref=src, dst_ref=ccw_ag.at[h],
                send_sem=ccw_ag_ssem.at[h], recv_sem=ccw_ag_rsem.at[h],
                device_id=(left,), device_id_type=pl.DeviceIdType.MESH,
            )

        ag_cw0 = ag_rdma(0, "cw")
        ag_cw0.start()
        send_waits.append(ag_cw0)

        g_bot = _gelu(reduced_bot)
        ccw_ag_my[...] = g_bot.astype(jnp.bfloat16)
        ag_ccw0 = ag_rdma(0, "ccw")
        ag_ccw0.start()
        send_waits.append(ag_ccw0)

        out_ref[pl.ds(q_cw * Q, H), :] = g_top
        out_ref[pl.ds(q_ccw * Q + H, H), :] = g_bot

        cw_ags = [ag_cw0]
        ccw_ags = [ag_ccw0]
        for h in range(N_DEV - 1):
            cw_ags[h].wait_recv()
            if h < N_DEV - 2:
                nxt = ag_rdma(h + 1, "cw")
                nxt.start()
                cw_ags.append(nxt)
                send_waits.append(nxt)
            o = lax.rem(i + (N_DEV - h), N_DEV)
            out_ref[pl.ds(o * Q, H), :] = cw_ag[h].astype(jnp.float32)

            ccw_ags[h].wait_recv()
            if h < N_DEV - 2:
                nxt = ag_rdma(h + 1, "ccw")
                nxt.start()
                ccw_ags.append(nxt)
                send_waits.append(nxt)
            o2 = lax.rem(i + h, N_DEV)
            out_ref[pl.ds(o2 * Q + H, H), :] = ccw_ag[h].astype(jnp.float32)

        for rdma in send_waits:
            rdma.wait_send()

    return pl.pallas_call(
        body,
        out_shape=jax.ShapeDtypeStruct((M, N), jnp.float32),
        in_specs=[
            pl.BlockSpec(memory_space=pltpu.VMEM),
            pl.BlockSpec(memory_space=pltpu.VMEM),
        ],
        out_specs=pl.BlockSpec(memory_space=pltpu.VMEM),
        scratch_shapes=[
            pltpu.VMEM((M, N), jnp.float32),
            pltpu.VMEM((N_DEV - 1, H, N), jnp.bfloat16),
            pltpu.VMEM((N_DEV - 1, H, N), jnp.bfloat16),
            pltpu.VMEM((N_DEV - 1, H, N), jnp.bfloat16),
            pltpu.VMEM((N_DEV - 1, H, N), jnp.bfloat16),
            pltpu.VMEM((H, N), jnp.bfloat16),
            pltpu.VMEM((N_DEV - 1, H, N), jnp.bfloat16),
            pltpu.VMEM((H, N), jnp.bfloat16),
            pltpu.VMEM((N_DEV - 1, H, N), jnp.bfloat16),
            pltpu.SemaphoreType.DMA((N_DEV - 1,)),
            pltpu.SemaphoreType.DMA((N_DEV - 1,)),
            pltpu.SemaphoreType.DMA((N_DEV - 1,)),
            pltpu.SemaphoreType.DMA((N_DEV - 1,)),
            pltpu.SemaphoreType.DMA((N_DEV - 1,)),
            pltpu.SemaphoreType.DMA((N_DEV - 1,)),
            pltpu.SemaphoreType.DMA((N_DEV - 1,)),
            pltpu.SemaphoreType.DMA((N_DEV - 1,)),
        ],
        compiler_params=pltpu.CompilerParams(collective_id=0),
    )(A, B)


# device time: 54612 ns/iter; 1.8955x vs baseline; 1.1900x over previous
import jax
import jax.numpy as jnp
from jax import lax
from jax.experimental import pallas as pl
from jax.experimental.pallas import tpu as pltpu

N_DEV = 4
SUB = 2


def _gelu(z):
    return 0.5 * z * (1.0 + jnp.tanh(0.7978845608 * (z + 0.044715 * z * z * z)))


def kernel(A, B):
    M, _ = A.shape
    _, N = B.shape
    Q = M // N_DEV
    H = Q // 2
    Hs = H // SUB

    def body(a_ref, b_ref, out_ref, part_ref, b_bf16,
             cw_send, cw_recv, ccw_send, ccw_recv,
             cw_ag_my, cw_ag, ccw_ag_my, ccw_ag,
             cw_rs_ssem, cw_rs_rsem, ccw_rs_ssem, ccw_rs_rsem,
             cw_ag_ssem, cw_ag_rsem, ccw_ag_ssem, ccw_ag_rsem):
        i = lax.axis_index("i")
        right = lax.rem(i + 1, N_DEV)
        left = lax.rem(i + (N_DEV - 1), N_DEV)

        cfg = {
            "cw": dict(send=cw_send, recv=cw_recv,
                       rs_ssem=cw_rs_ssem, rs_rsem=cw_rs_rsem,
                       ag_my=cw_ag_my, ag=cw_ag,
                       ag_ssem=cw_ag_ssem, ag_rsem=cw_ag_rsem,
                       dev=right, base=0),
            "ccw": dict(send=ccw_send, recv=ccw_recv,
                        rs_ssem=ccw_rs_ssem, rs_rsem=ccw_rs_rsem,
                        ag_my=ccw_ag_my, ag=ccw_ag,
                        ag_ssem=ccw_ag_ssem, ag_rsem=ccw_ag_rsem,
                        dev=left, base=H),
        }

        def rs_add_chunk(d, s):
            if d == "cw":
                return lax.rem(i + (2 * N_DEV - s - 1), N_DEV)
            return lax.rem(i + s + 1, N_DEV)

        def own_chunk(d):
            if d == "cw":
                return lax.rem(i + 1, N_DEV)
            return lax.rem(i + (N_DEV - 1), N_DEV)

        def ag_origin(d, h):
            if d == "cw":
                return lax.rem(i + (N_DEV - h), N_DEV)
            return lax.rem(i + h, N_DEV)

        def part_sub(d, c, j):
            return part_ref[pl.ds(c * Q + cfg[d]["base"] + j * Hs, Hs), :]

        def rs_rdma(d, s, j):
            c = cfg[d]
            return pltpu.make_async_remote_copy(
                src_ref=c["send"].at[s, pl.ds(j * Hs, Hs)],
                dst_ref=c["recv"].at[s, pl.ds(j * Hs, Hs)],
                send_sem=c["rs_ssem"].at[s, j],
                recv_sem=c["rs_rsem"].at[s, j],
                device_id=(c["dev"],),
                device_id_type=pl.DeviceIdType.MESH,
            )

        def ag_rdma(d, h, j):
            c = cfg[d]
            src = c["ag_my"] if h == 0 else c["ag"].at[h - 1]
            return pltpu.make_async_remote_copy(
                src_ref=src.at[pl.ds(j * Hs, Hs)],
                dst_ref=c["ag"].at[h, pl.ds(j * Hs, Hs)],
                send_sem=c["ag_ssem"].at[h, j],
                recv_sem=c["ag_rsem"].at[h, j],
                device_id=(c["dev"],),
                device_id_type=pl.DeviceIdType.MESH,
            )

        barrier = pltpu.get_barrier_semaphore()
        for nbr in (left, right):
            pl.semaphore_signal(
                barrier, inc=1,
                device_id=(nbr,), device_id_type=pl.DeviceIdType.MESH,
            )
        pl.semaphore_wait(barrier, 2)

        b_bf16[...] = b_ref[...].astype(jnp.bfloat16)

        def compute_chunk(c):
            part_ref[pl.ds(c * Q, Q), :] = jnp.dot(
                a_ref[pl.ds(c * Q, Q), :].astype(jnp.bfloat16),
                b_bf16[...],
                preferred_element_type=jnp.float32,
            )

        compute_chunk(i)

        send_waits = []
        rs = {d: [[None] * SUB for _ in range(N_DEV - 1)] for d in cfg}
        for d in cfg:
            cfg[d]["send"][0] = part_ref[
                pl.ds(i * Q + cfg[d]["base"], H), :].astype(jnp.bfloat16)
            for j in range(SUB):
                r = rs_rdma(d, 0, j)
                r.start()
                rs[d][0][j] = r
                send_waits.append(r)

        compute_chunk(lax.rem(i + (N_DEV - 1), N_DEV))
        compute_chunk(lax.rem(i + 1, N_DEV))
        compute_chunk(lax.rem(i + 2, N_DEV))

        reduced = {d: [None] * SUB for d in cfg}
        for s in range(N_DEV - 1):
            for j in range(SUB):
                for d in cfg:
                    rs[d][s][j].wait_recv()
                    summed = (
                        cfg[d]["recv"][s, pl.ds(j * Hs, Hs)].astype(jnp.float32)
                        + part_sub(d, rs_add_chunk(d, s), j)
                    )
                    if s < N_DEV - 2:
                        cfg[d]["send"][s + 1, pl.ds(j * Hs, Hs)] = (
                            summed.astype(jnp.bfloat16))
                        nxt = rs_rdma(d, s + 1, j)
                        nxt.start()
                        rs[d][s + 1][j] = nxt
                        send_waits.append(nxt)
                    else:
                        reduced[d][j] = summed

        ag = {d: [[None] * SUB for _ in range(N_DEV - 1)] for d in cfg}
        g_out = {d: [None] * SUB for d in cfg}
        for j in range(SUB):
            for d in cfg:
                g = _gelu(reduced[d][j])
                g_out[d][j] = g
                cfg[d]["ag_my"][pl.ds(j * Hs, Hs), :] = g.astype(jnp.bfloat16)
                r = ag_rdma(d, 0, j)
                r.start()
                ag[d][0][j] = r
                send_waits.append(r)
        for d in cfg:
            q = own_chunk(d)
            for j in range(SUB):
                out_ref[pl.ds(q * Q + cfg[d]["base"] + j * Hs, Hs), :] = (
                    g_out[d][j])

        for h in range(N_DEV - 1):
            for j in range(SUB):
                stores = []
                for d in cfg:
                    ag[d][h][j].wait_recv()
                    if h < N_DEV - 2:
                        nxt = ag_rdma(d, h + 1, j)
                        nxt.start()
                        ag[d][h + 1][j] = nxt
                        send_waits.append(nxt)
                    stores.append(d)
                for d in stores:
                    o = ag_origin(d, h)
                    out_ref[pl.ds(o * Q + cfg[d]["base"] + j * Hs, Hs), :] = (
                        cfg[d]["ag"][h, pl.ds(j * Hs, Hs)].astype(jnp.float32))

        for rdma in send_waits:
            rdma.wait_send()

    return pl.pallas_call(
        body,
        out_shape=jax.ShapeDtypeStruct((M, N), jnp.float32),
        in_specs=[
            pl.BlockSpec(memory_space=pltpu.VMEM),
            pl.BlockSpec(memory_space=pltpu.VMEM),
        ],
        out_specs=pl.BlockSpec(memory_space=pltpu.VMEM),
        scratch_shapes=[
            pltpu.VMEM((M, N), jnp.float32),
            pltpu.VMEM((B.shape[0], N), jnp.bfloat16),
            pltpu.VMEM((N_DEV - 1, H, N), jnp.bfloat16),
            pltpu.VMEM((N_DEV - 1, H, N), jnp.bfloat16),
            pltpu.VMEM((N_DEV - 1, H, N), jnp.bfloat16),
            pltpu.VMEM((N_DEV - 1, H, N), jnp.bfloat16),
            pltpu.VMEM((H, N), jnp.bfloat16),
            pltpu.VMEM((N_DEV - 1, H, N), jnp.bfloat16),
            pltpu.VMEM((H, N), jnp.bfloat16),
            pltpu.VMEM((N_DEV - 1, H, N), jnp.bfloat16),
            pltpu.SemaphoreType.DMA((N_DEV - 1, SUB)),
            pltpu.SemaphoreType.DMA((N_DEV - 1, SUB)),
            pltpu.SemaphoreType.DMA((N_DEV - 1, SUB)),
            pltpu.SemaphoreType.DMA((N_DEV - 1, SUB)),
            pltpu.SemaphoreType.DMA((N_DEV - 1, SUB)),
            pltpu.SemaphoreType.DMA((N_DEV - 1, SUB)),
            pltpu.SemaphoreType.DMA((N_DEV - 1, SUB)),
            pltpu.SemaphoreType.DMA((N_DEV - 1, SUB)),
        ],
        compiler_params=pltpu.CompilerParams(collective_id=0),
    )(A, B)
